# baseline (speedup 1.0000x reference)
import functools

import jax
import jax.numpy as jnp
from jax import lax
from jax.experimental import pallas as pl
from jax.experimental.pallas import tpu as pltpu
from jax.experimental.pallas import tpu_sc as plsc

DIM = 128
NSEG = 3
EPS = 1e-5
LANES = 16
NJ = DIM // LANES
NC = 2
NS = 16
NW = NC * NS
C = 256
CH = 128


def _build(B, S, V):
    rows_total = B * S
    rows_per_tile = rows_total // NW
    nchunk = rows_per_tile // C
    chunks_per_seq = S // C
    inv_d = 1.0 / DIM
    niter = (nchunk + 1 + 2) // 3
    klast = niter * 3 - 1

    mesh = plsc.VectorSubcoreMesh(core_axis_name="c", subcore_axis_name="s")

    @functools.partial(
        pl.kernel,
        mesh=mesh,
        out_type=jax.ShapeDtypeStruct((rows_total, DIM), jnp.float32),
        compiler_params=pltpu.CompilerParams(needs_layout_passes=False),
        scratch_types=(
            [pltpu.VMEM((DIM,), jnp.float32)] * 2
            + [pltpu.VMEM((C,), jnp.int32)] * 3
            + [pltpu.VMEM((C,), jnp.int32)] * 3
            + [pltpu.VMEM((C,), jnp.int32)] * 3
            + [pltpu.VMEM((C, DIM), jnp.float32)] * 3
            + [pltpu.SemaphoreType.DMA] * 15
        ),
    )
    def sc_kernel(seq_hbm, lab_hbm, tok_hbm, comb_hbm, gamma_hbm, beta_hbm,
                  out_hbm, gamma_v, beta_v, idx0, idx1, idx2, lab0, lab1,
                  lab2, il0, il1, il2, buf0, buf1, buf2, isem0, isem1, isem2,
                  lsem0, lsem1, lsem2, g1s0, g1s1, g1s2, g2s0, g2s1, g2s2,
                  osem0, osem1, osem2):
        wid = lax.axis_index("s") * NC + lax.axis_index("c")
        row_base = wid * rows_per_tile
        lane = lax.iota(jnp.int32, LANES)
        idx = [idx0, idx1, idx2]
        labs = [lab0, lab1, lab2]
        ils = [il0, il1, il2]
        buf = [buf0, buf1, buf2]
        isem = [isem0, isem1, isem2]
        lsem = [lsem0, lsem1, lsem2]
        g1sem = [g1s0, g1s1, g1s2]
        g2sem = [g2s0, g2s1, g2s2]
        osem = [osem0, osem1, osem2]

        pltpu.sync_copy(gamma_hbm, gamma_v)
        pltpu.sync_copy(beta_hbm, beta_v)
        gammas = [gamma_v[pl.ds(j * LANES, LANES)] for j in range(NJ)]
        betas = [beta_v[pl.ds(j * LANES, LANES)] for j in range(NJ)]

        def chunk_base(k):
            return row_base + jnp.minimum(k, nchunk - 1) * C

        def chunk_p0(k):
            return lax.rem(jnp.minimum(k, nchunk - 1), chunks_per_seq) * C

        def ifetch_start(k, slot):
            base = chunk_base(k)
            pltpu.make_async_copy(seq_hbm.at[pl.ds(base, C)], idx[slot],
                                  isem[slot]).start()
            pltpu.make_async_copy(lab_hbm.at[pl.ds(base, C)], labs[slot],
                                  lsem[slot]).start()

        def iwait(slot):
            pltpu.make_async_copy(seq_hbm.at[pl.ds(row_base, C)], idx[slot],
                                  isem[slot]).wait()

        def lwait(slot):
            pltpu.make_async_copy(lab_hbm.at[pl.ds(row_base, C)], labs[slot],
                                  lsem[slot]).wait()

        def build_il(k, slot):
            p0 = chunk_p0(k)
            lab_s = labs[slot]
            il_s = ils[slot]
            for g in range(C // LANES):
                lv = lab_s[pl.ds(g * LANES, LANES)]
                il_s[pl.ds(g * LANES, LANES)] = lv * S + (p0 + g * LANES + lane)

        def g1_start(slot):
            for h in range(C // CH):
                pltpu.make_async_copy(
                    tok_hbm.at[idx[slot].at[pl.ds(h * CH, CH)]],
                    buf[slot].at[pl.ds(h * CH, CH)], g1sem[slot]).start()

        def g1_wait(slot):
            for h in range(C // CH):
                pltpu.make_async_copy(
                    tok_hbm.at[idx[slot].at[pl.ds(h * CH, CH)]],
                    buf[slot].at[pl.ds(h * CH, CH)], g1sem[slot]).wait()

        def g2_start(slot):
            for h in range(C // CH):
                pltpu.make_async_copy(
                    comb_hbm.at[ils[slot].at[pl.ds(h * CH, CH)]],
                    buf[slot].at[pl.ds(h * CH, CH)], g2sem[slot]).start(add=True)

        def g2_wait(slot):
            for h in range(C // CH):
                pltpu.make_async_copy(
                    comb_hbm.at[ils[slot].at[pl.ds(h * CH, CH)]],
                    buf[slot].at[pl.ds(h * CH, CH)], g2sem[slot]).wait()

        def out_start(k, slot):
            pltpu.make_async_copy(buf[slot], out_hbm.at[pl.ds(chunk_base(k), C)],
                                  osem[slot]).start()

        def out_wait(slot):
            pltpu.make_async_copy(buf[slot], out_hbm.at[pl.ds(row_base, C)],
                                  osem[slot]).wait()

        def compute(slot):
            buf_s = buf[slot]

            @plsc.parallel_loop(0, C, unroll=8)
            def row_body(r):
                s = None
                q = None
                xs = []
                for j in range(NJ):
                    x = buf_s[r, pl.ds(j * LANES, LANES)]
                    xs.append(x)
                    s = x if s is None else s + x
                    q = x * x if q is None else q + x * x
                mean = jnp.full((LANES,), jnp.sum(s), jnp.float32) * inv_d
                var = (jnp.full((LANES,), jnp.sum(q), jnp.float32) * inv_d
                       - mean * mean)
                ve = var + EPS
                seed = jnp.int32(0x5F3759DF) - (plsc.bitcast(ve, jnp.int32) >> 1)
                y = plsc.bitcast(seed, jnp.float32)
                for _ in range(2):
                    y = y * (1.5 - 0.5 * ve * y * y)
                for j in range(NJ):
                    out = (xs[j] - mean) * y * gammas[j] + betas[j]
                    buf_s[r, pl.ds(j * LANES, LANES)] = out

        ifetch_start(0, 0)
        ifetch_start(1, 1)
        ifetch_start(2, 2)
        iwait(0)
        g1_start(0)
        lwait(0)
        build_il(0, 0)
        iwait(1)
        g1_start(1)
        g1_wait(0)
        g2_start(0)

        def body(k3, _):
            for par in range(3):
                k = k3 * 3 + par
                s0 = par
                s1 = (par + 1) % 3
                s2 = (par + 2) % 3

                def wait_buf(s2=s2):
                    out_wait(s2)

                if par == 0:
                    pl.when(k > 0)(wait_buf)
                else:
                    wait_buf()
                iwait(s2)
                g1_start(s2)
                lwait(s1)
                build_il(k + 1, s1)
                g1_wait(s1)
                g2_start(s1)
                g2_wait(s0)
                compute(s0)
                out_start(k, s0)
                ifetch_start(k + 3, s0)
            return 0

        lax.fori_loop(0, niter, body, 0)

        out_wait(klast % 3)
        g2_wait((klast + 1) % 3)
        g1_wait((klast + 2) % 3)
        iwait(klast % 3)
        lwait((klast + 2) % 3)
        lwait(klast % 3)

    return sc_kernel


def kernel(sequence, segment_label, token_table, seg_table, gamma, beta, pe):
    B, S = sequence.shape
    V = token_table.shape[0]
    seq = sequence.reshape(-1).astype(jnp.int32)
    lab = segment_label.reshape(-1).astype(jnp.int32)
    comb = (seg_table[:, None, :] + pe[None, :S, :]).reshape(NSEG * S, DIM)
    out = _build(B, S, V)(seq, lab, token_table, comb, gamma, beta)
    return out.reshape(B, S, DIM)


# speedup vs baseline: 1.1467x; 1.1467x over previous
import functools

import jax
import jax.numpy as jnp
from jax import lax
from jax.experimental import pallas as pl
from jax.experimental.pallas import tpu as pltpu
from jax.experimental.pallas import tpu_sc as plsc

DIM = 128
NSEG = 3
EPS = 1e-5
LANES = 16
NJ = DIM // LANES
NC = 2
NS = 16
NW = NC * NS
C = 128


def _build(B, S, V):
    rows_total = B * S
    rows_per_tile = rows_total // NW
    nchunk = rows_per_tile // C
    chunks_per_seq = S // C
    inv_d = 1.0 / DIM
    niter = (nchunk + 1 + 2) // 3
    klast = niter * 3 - 1

    mesh = plsc.VectorSubcoreMesh(core_axis_name="c", subcore_axis_name="s")

    @functools.partial(
        pl.kernel,
        mesh=mesh,
        out_type=jax.ShapeDtypeStruct((rows_total, DIM), jnp.float32),
        compiler_params=pltpu.CompilerParams(needs_layout_passes=False),
        scratch_types=(
            [pltpu.VMEM((DIM,), jnp.float32)] * 2
            + [pltpu.VMEM((C,), jnp.int32)] * 3
            + [pltpu.VMEM((C,), jnp.int32)] * 3
            + [pltpu.VMEM((C,), jnp.int32)] * 3
            + [pltpu.VMEM((C, DIM), jnp.float32)] * 3
            + [pltpu.VMEM_SHARED((NSEG * S, DIM), jnp.float32)]
            + [pltpu.SemaphoreType.DMA] * 15
        ),
    )
    def sc_kernel(seq_hbm, lab_hbm, tok_hbm, comb_hbm, gamma_hbm, beta_hbm,
                  out_hbm, gamma_v, beta_v, idx0, idx1, idx2, lab0, lab1,
                  lab2, il0, il1, il2, buf0, buf1, buf2, comb_sp, isem0,
                  isem1, isem2,
                  lsem0, lsem1, lsem2, g1s0, g1s1, g1s2, g2s0, g2s1, g2s2,
                  osem0, osem1, osem2):
        wid = lax.axis_index("s") * NC + lax.axis_index("c")
        row_base = wid * rows_per_tile
        lane = lax.iota(jnp.int32, LANES)
        idx = [idx0, idx1, idx2]
        labs = [lab0, lab1, lab2]
        ils = [il0, il1, il2]
        buf = [buf0, buf1, buf2]
        isem = [isem0, isem1, isem2]
        lsem = [lsem0, lsem1, lsem2]
        g1sem = [g1s0, g1s1, g1s2]
        g2sem = [g2s0, g2s1, g2s2]
        osem = [osem0, osem1, osem2]

        @pl.when(lax.axis_index("s") == 0)
        def _stage_comb():
            pltpu.sync_copy(comb_hbm, comb_sp)

        plsc.subcore_barrier()
        pltpu.sync_copy(gamma_hbm, gamma_v)
        pltpu.sync_copy(beta_hbm, beta_v)
        gammas = [gamma_v[pl.ds(j * LANES, LANES)] for j in range(NJ)]
        betas = [beta_v[pl.ds(j * LANES, LANES)] for j in range(NJ)]

        def chunk_base(k):
            return row_base + jnp.minimum(k, nchunk - 1) * C

        def chunk_p0(k):
            return lax.rem(jnp.minimum(k, nchunk - 1), chunks_per_seq) * C

        def ifetch_start(k, slot):
            base = chunk_base(k)
            pltpu.make_async_copy(seq_hbm.at[pl.ds(base, C)], idx[slot],
                                  isem[slot]).start()
            pltpu.make_async_copy(lab_hbm.at[pl.ds(base, C)], labs[slot],
                                  lsem[slot]).start()

        def iwait(slot):
            pltpu.make_async_copy(seq_hbm.at[pl.ds(row_base, C)], idx[slot],
                                  isem[slot]).wait()

        def lwait(slot):
            pltpu.make_async_copy(lab_hbm.at[pl.ds(row_base, C)], labs[slot],
                                  lsem[slot]).wait()

        def build_il(k, slot):
            p0 = chunk_p0(k)
            lab_s = labs[slot]
            il_s = ils[slot]
            for g in range(C // LANES):
                lv = lab_s[pl.ds(g * LANES, LANES)]
                il_s[pl.ds(g * LANES, LANES)] = lv * S + (p0 + g * LANES + lane)

        def g1_start(slot):
            pltpu.make_async_copy(tok_hbm.at[idx[slot]], buf[slot],
                                  g1sem[slot]).start()

        def g1_wait(slot):
            pltpu.make_async_copy(tok_hbm.at[idx[slot]], buf[slot],
                                  g1sem[slot]).wait()

        def g2_start(slot):
            pltpu.make_async_copy(comb_sp.at[ils[slot]], buf[slot],
                                  g2sem[slot]).start(add=True)

        def g2_wait(slot):
            pltpu.make_async_copy(comb_sp.at[ils[slot]], buf[slot],
                                  g2sem[slot]).wait()

        def out_start(k, slot):
            pltpu.make_async_copy(buf[slot], out_hbm.at[pl.ds(chunk_base(k), C)],
                                  osem[slot]).start()

        def out_wait(slot):
            pltpu.make_async_copy(buf[slot], out_hbm.at[pl.ds(row_base, C)],
                                  osem[slot]).wait()

        def compute(slot):
            buf_s = buf[slot]

            @plsc.parallel_loop(0, C, unroll=8)
            def row_body(r):
                s = None
                q = None
                xs = []
                for j in range(NJ):
                    x = buf_s[r, pl.ds(j * LANES, LANES)]
                    xs.append(x)
                    s = x if s is None else s + x
                    q = x * x if q is None else q + x * x
                mean = jnp.full((LANES,), jnp.sum(s), jnp.float32) * inv_d
                var = (jnp.full((LANES,), jnp.sum(q), jnp.float32) * inv_d
                       - mean * mean)
                ve = var + EPS
                seed = jnp.int32(0x5F3759DF) - (plsc.bitcast(ve, jnp.int32) >> 1)
                y = plsc.bitcast(seed, jnp.float32)
                for _ in range(2):
                    y = y * (1.5 - 0.5 * ve * y * y)
                for j in range(NJ):
                    out = (xs[j] - mean) * y * gammas[j] + betas[j]
                    buf_s[r, pl.ds(j * LANES, LANES)] = out

        ifetch_start(0, 0)
        ifetch_start(1, 1)
        ifetch_start(2, 2)
        iwait(0)
        g1_start(0)
        lwait(0)
        build_il(0, 0)
        iwait(1)
        g1_start(1)
        g1_wait(0)
        g2_start(0)

        def body(k3, _):
            for par in range(3):
                k = k3 * 3 + par
                s0 = par
                s1 = (par + 1) % 3
                s2 = (par + 2) % 3

                def wait_buf(s2=s2):
                    out_wait(s2)

                if par == 0:
                    pl.when(k > 0)(wait_buf)
                else:
                    wait_buf()
                iwait(s2)
                g1_start(s2)
                lwait(s1)
                build_il(k + 1, s1)
                g1_wait(s1)
                g2_start(s1)
                g2_wait(s0)
                compute(s0)
                out_start(k, s0)
                ifetch_start(k + 3, s0)
            return 0

        lax.fori_loop(0, niter, body, 0)

        out_wait(klast % 3)
        g2_wait((klast + 1) % 3)
        g1_wait((klast + 2) % 3)
        iwait(klast % 3)
        lwait((klast + 2) % 3)
        lwait(klast % 3)

    return sc_kernel


def kernel(sequence, segment_label, token_table, seg_table, gamma, beta, pe):
    B, S = sequence.shape
    V = token_table.shape[0]
    seq = sequence.reshape(-1).astype(jnp.int32)
    lab = segment_label.reshape(-1).astype(jnp.int32)
    comb = (seg_table[:, None, :] + pe[None, :S, :]).reshape(NSEG * S, DIM)
    out = _build(B, S, V)(seq, lab, token_table, comb, gamma, beta)
    return out.reshape(B, S, DIM)


# speedup vs baseline: 1.3064x; 1.1392x over previous
import functools

import jax
import jax.numpy as jnp
from jax import lax
from jax.experimental import pallas as pl
from jax.experimental.pallas import tpu as pltpu
from jax.experimental.pallas import tpu_sc as plsc

DIM = 128
NSEG = 3
EPS = 1e-5
LANES = 16
NJ = DIM // LANES
NC = 2
NS = 16
NW = NC * NS
C = 128


def _build(B, S, V):
    rows_total = B * S
    rows_per_tile = rows_total // NW
    nchunk = rows_per_tile // C
    chunks_per_seq = S // C
    inv_d = 1.0 / DIM
    niter = (nchunk + 1 + 2) // 3
    klast = niter * 3 - 1

    mesh = plsc.VectorSubcoreMesh(core_axis_name="c", subcore_axis_name="s")

    @functools.partial(
        pl.kernel,
        mesh=mesh,
        out_type=jax.ShapeDtypeStruct((rows_total, DIM), jnp.float32),
        compiler_params=pltpu.CompilerParams(needs_layout_passes=False),
        scratch_types=(
            [pltpu.VMEM((DIM,), jnp.float32)] * 2
            + [pltpu.VMEM((C,), jnp.int32)] * 3
            + [pltpu.VMEM((C,), jnp.int32)] * 3
            + [pltpu.VMEM((C,), jnp.int32)] * 3
            + [pltpu.VMEM((C, DIM), jnp.float32)] * 3
            + [pltpu.VMEM_SHARED((NSEG * S, DIM), jnp.float32)]
            + [pltpu.SemaphoreType.DMA] * 15
        ),
    )
    def sc_kernel(seq_hbm, lab_hbm, tok_hbm, comb_hbm, gamma_hbm, beta_hbm,
                  out_hbm, gamma_v, beta_v, idx0, idx1, idx2, lab0, lab1,
                  lab2, il0, il1, il2, buf0, buf1, buf2, comb_sp, isem0,
                  isem1, isem2,
                  lsem0, lsem1, lsem2, g1s0, g1s1, g1s2, g2s0, g2s1, g2s2,
                  osem0, osem1, osem2):
        wid = lax.axis_index("s") * NC + lax.axis_index("c")
        row_base = wid * rows_per_tile
        lane = lax.iota(jnp.int32, LANES)
        idx = [idx0, idx1, idx2]
        labs = [lab0, lab1, lab2]
        ils = [il0, il1, il2]
        buf = [buf0, buf1, buf2]
        isem = [isem0, isem1, isem2]
        lsem = [lsem0, lsem1, lsem2]
        g1sem = [g1s0, g1s1, g1s2]
        g2sem = [g2s0, g2s1, g2s2]
        osem = [osem0, osem1, osem2]

        @pl.when(lax.axis_index("s") == 0)
        def _stage_comb():
            pltpu.sync_copy(comb_hbm, comb_sp)

        plsc.subcore_barrier()
        pltpu.sync_copy(gamma_hbm, gamma_v)
        pltpu.sync_copy(beta_hbm, beta_v)
        gammas = [gamma_v[pl.ds(j * LANES, LANES)] for j in range(NJ)]
        betas = [beta_v[pl.ds(j * LANES, LANES)] for j in range(NJ)]

        def chunk_base(k):
            return row_base + jnp.minimum(k, nchunk - 1) * C

        def chunk_p0(k):
            return lax.rem(jnp.minimum(k, nchunk - 1), chunks_per_seq) * C

        def ifetch_start(k, slot):
            base = chunk_base(k)
            pltpu.make_async_copy(seq_hbm.at[pl.ds(base, C)], idx[slot],
                                  isem[slot]).start()
            pltpu.make_async_copy(lab_hbm.at[pl.ds(base, C)], labs[slot],
                                  lsem[slot]).start()

        def iwait(slot):
            pltpu.make_async_copy(seq_hbm.at[pl.ds(row_base, C)], idx[slot],
                                  isem[slot]).wait()

        def lwait(slot):
            pltpu.make_async_copy(lab_hbm.at[pl.ds(row_base, C)], labs[slot],
                                  lsem[slot]).wait()

        def build_il(k, slot):
            p0 = chunk_p0(k)
            lab_s = labs[slot]
            il_s = ils[slot]
            for g in range(C // LANES):
                lv = lab_s[pl.ds(g * LANES, LANES)]
                il_s[pl.ds(g * LANES, LANES)] = lv * S + (p0 + g * LANES + lane)

        def g1_start(slot):
            pltpu.make_async_copy(tok_hbm.at[idx[slot]], buf[slot],
                                  g1sem[slot]).start()

        def g1_wait(slot):
            pltpu.make_async_copy(tok_hbm.at[idx[slot]], buf[slot],
                                  g1sem[slot]).wait()

        def g2_start(slot):
            pltpu.make_async_copy(comb_sp.at[ils[slot]], buf[slot],
                                  g2sem[slot]).start(add=True)

        def g2_wait(slot):
            pltpu.make_async_copy(comb_sp.at[ils[slot]], buf[slot],
                                  g2sem[slot]).wait()

        def out_start(k, slot):
            pltpu.make_async_copy(buf[slot], out_hbm.at[pl.ds(chunk_base(k), C)],
                                  osem[slot]).start()

        def out_wait(slot):
            pltpu.make_async_copy(buf[slot], out_hbm.at[pl.ds(row_base, C)],
                                  osem[slot]).wait()

        def compute(slot):
            buf_s = buf[slot]

            @plsc.parallel_loop(0, C, unroll=4)
            def row_body(r):
                s = None
                q = None
                xs = []
                for j in range(NJ):
                    x = buf_s[r, pl.ds(j * LANES, LANES)]
                    xs.append(x)
                    s = x if s is None else s + x
                    q = x * x if q is None else q + x * x
                mean = jnp.full((LANES,), jnp.sum(s), jnp.float32) * inv_d
                var = (jnp.full((LANES,), jnp.sum(q), jnp.float32) * inv_d
                       - mean * mean)
                ve = var + EPS
                seed = jnp.int32(0x5F3759DF) - (plsc.bitcast(ve, jnp.int32) >> 1)
                y = plsc.bitcast(seed, jnp.float32)
                for _ in range(2):
                    y = y * (1.5 - 0.5 * ve * y * y)
                for j in range(NJ):
                    out = (xs[j] - mean) * y * gammas[j] + betas[j]
                    buf_s[r, pl.ds(j * LANES, LANES)] = out

        ifetch_start(0, 0)
        ifetch_start(1, 1)
        ifetch_start(2, 2)
        iwait(0)
        g1_start(0)
        lwait(0)
        build_il(0, 0)
        iwait(1)
        g1_start(1)
        g1_wait(0)
        g2_start(0)

        def body(k3, _):
            for par in range(3):
                k = k3 * 3 + par
                s0 = par
                s1 = (par + 1) % 3
                s2 = (par + 2) % 3

                def wait_buf(s2=s2):
                    out_wait(s2)

                if par == 0:
                    pl.when(k > 0)(wait_buf)
                else:
                    wait_buf()
                iwait(s2)
                g1_start(s2)
                lwait(s1)
                build_il(k + 1, s1)
                g1_wait(s1)
                g2_start(s1)
                g2_wait(s0)
                compute(s0)
                out_start(k, s0)
                ifetch_start(k + 3, s0)
            return 0

        lax.fori_loop(0, niter, body, 0)

        out_wait(klast % 3)
        g2_wait((klast + 1) % 3)
        g1_wait((klast + 2) % 3)
        iwait(klast % 3)
        lwait((klast + 2) % 3)
        lwait(klast % 3)

    return sc_kernel


def kernel(sequence, segment_label, token_table, seg_table, gamma, beta, pe):
    B, S = sequence.shape
    V = token_table.shape[0]
    seq = sequence.reshape(-1).astype(jnp.int32)
    lab = segment_label.reshape(-1).astype(jnp.int32)
    comb = (seg_table[:, None, :] + pe[None, :S, :]).reshape(NSEG * S, DIM)
    out = _build(B, S, V)(seq, lab, token_table, comb, gamma, beta)
    return out.reshape(B, S, DIM)


# speedup vs baseline: 2.0639x; 1.5799x over previous
import functools

import jax
import jax.numpy as jnp
from jax import lax
from jax.experimental import pallas as pl
from jax.experimental.pallas import tpu as pltpu
from jax.experimental.pallas import tpu_sc as plsc

DIM = 128
NSEG = 3
EPS = 1e-5
LANES = 16
NJ = DIM // LANES
NC = 2
NS = 16
NW = NC * NS
C = 128


def _build(B, S, V):
    rows_total = B * S
    rows_per_tile = rows_total // NW
    nchunk = rows_per_tile // C
    chunks_per_seq = S // C
    inv_d = 1.0 / DIM
    niter = (nchunk + 1 + 2) // 3
    klast = niter * 3 - 1

    mesh = plsc.VectorSubcoreMesh(core_axis_name="c", subcore_axis_name="s")

    @functools.partial(
        pl.kernel,
        mesh=mesh,
        out_type=jax.ShapeDtypeStruct((rows_total, DIM), jnp.float32),
        compiler_params=pltpu.CompilerParams(needs_layout_passes=False),
        scratch_types=(
            [pltpu.VMEM((DIM,), jnp.float32)] * 2
            + [pltpu.VMEM((C,), jnp.int32)] * 3
            + [pltpu.VMEM((C,), jnp.int32)] * 3
            + [pltpu.VMEM((C,), jnp.int32)] * 3
            + [pltpu.VMEM((C, DIM), jnp.float32)] * 3
            + [pltpu.VMEM_SHARED((NSEG * S, DIM), jnp.float32)]
            + [pltpu.SemaphoreType.DMA] * 15
        ),
    )
    def sc_kernel(seq_hbm, lab_hbm, tok_hbm, comb_hbm, gamma_hbm, beta_hbm,
                  out_hbm, gamma_v, beta_v, idx0, idx1, idx2, lab0, lab1,
                  lab2, il0, il1, il2, buf0, buf1, buf2, comb_sp, isem0,
                  isem1, isem2,
                  lsem0, lsem1, lsem2, g1s0, g1s1, g1s2, g2s0, g2s1, g2s2,
                  osem0, osem1, osem2):
        wid = lax.axis_index("s") * NC + lax.axis_index("c")
        row_base = wid * rows_per_tile
        lane = lax.iota(jnp.int32, LANES)
        idx = [idx0, idx1, idx2]
        labs = [lab0, lab1, lab2]
        ils = [il0, il1, il2]
        buf = [buf0, buf1, buf2]
        isem = [isem0, isem1, isem2]
        lsem = [lsem0, lsem1, lsem2]
        g1sem = [g1s0, g1s1, g1s2]
        g2sem = [g2s0, g2s1, g2s2]
        osem = [osem0, osem1, osem2]

        @pl.when(lax.axis_index("s") == 0)
        def _stage_comb():
            pltpu.sync_copy(comb_hbm, comb_sp)

        plsc.subcore_barrier()
        pltpu.sync_copy(gamma_hbm, gamma_v)
        pltpu.sync_copy(beta_hbm, beta_v)
        gammas = [gamma_v[pl.ds(j * LANES, LANES)] for j in range(NJ)]
        betas = [beta_v[pl.ds(j * LANES, LANES)] for j in range(NJ)]

        def chunk_base(k):
            return row_base + jnp.minimum(k, nchunk - 1) * C

        def chunk_p0(k):
            return lax.rem(jnp.minimum(k, nchunk - 1), chunks_per_seq) * C

        def ifetch_start(k, slot):
            base = chunk_base(k)
            pltpu.make_async_copy(seq_hbm.at[pl.ds(base, C)], idx[slot],
                                  isem[slot]).start()
            pltpu.make_async_copy(lab_hbm.at[pl.ds(base, C)], labs[slot],
                                  lsem[slot]).start()

        def iwait(slot):
            pltpu.make_async_copy(seq_hbm.at[pl.ds(row_base, C)], idx[slot],
                                  isem[slot]).wait()

        def lwait(slot):
            pltpu.make_async_copy(lab_hbm.at[pl.ds(row_base, C)], labs[slot],
                                  lsem[slot]).wait()

        def build_il(k, slot):
            p0 = chunk_p0(k)
            lab_s = labs[slot]
            il_s = ils[slot]
            for g in range(C // LANES):
                lv = lab_s[pl.ds(g * LANES, LANES)]
                il_s[pl.ds(g * LANES, LANES)] = lv * S + (p0 + g * LANES + lane)

        def g1_start(slot):
            pltpu.make_async_copy(tok_hbm.at[idx[slot]], buf[slot],
                                  g1sem[slot]).start()

        def g1_wait(slot):
            pltpu.make_async_copy(tok_hbm.at[idx[slot]], buf[slot],
                                  g1sem[slot]).wait()

        def g2_start(slot):
            pltpu.make_async_copy(comb_sp.at[ils[slot]], buf[slot],
                                  g2sem[slot]).start(add=True)

        def g2_wait(slot):
            pltpu.make_async_copy(comb_sp.at[ils[slot]], buf[slot],
                                  g2sem[slot]).wait()

        def out_start(k, slot):
            pltpu.make_async_copy(buf[slot], out_hbm.at[pl.ds(chunk_base(k), C)],
                                  osem[slot]).start()

        def out_wait(slot):
            pltpu.make_async_copy(buf[slot], out_hbm.at[pl.ds(row_base, C)],
                                  osem[slot]).wait()

        def compute(slot):
            buf_s = buf[slot]

            @plsc.parallel_loop(0, C, unroll=2)
            def row_body(r):
                s = None
                q = None
                xs = []
                for j in range(NJ):
                    x = buf_s[r, pl.ds(j * LANES, LANES)]
                    xs.append(x)
                    s = x if s is None else s + x
                    q = x * x if q is None else q + x * x
                mean = jnp.full((LANES,), jnp.sum(s), jnp.float32) * inv_d
                var = (jnp.full((LANES,), jnp.sum(q), jnp.float32) * inv_d
                       - mean * mean)
                ve = var + EPS
                seed = jnp.int32(0x5F3759DF) - (plsc.bitcast(ve, jnp.int32) >> 1)
                y = plsc.bitcast(seed, jnp.float32)
                for _ in range(2):
                    y = y * (1.5 - 0.5 * ve * y * y)
                for j in range(NJ):
                    out = (xs[j] - mean) * y * gammas[j] + betas[j]
                    buf_s[r, pl.ds(j * LANES, LANES)] = out

        ifetch_start(0, 0)
        ifetch_start(1, 1)
        ifetch_start(2, 2)
        iwait(0)
        g1_start(0)
        lwait(0)
        build_il(0, 0)
        iwait(1)
        g1_start(1)
        g1_wait(0)
        g2_start(0)

        def body(k3, _):
            for par in range(3):
                k = k3 * 3 + par
                s0 = par
                s1 = (par + 1) % 3
                s2 = (par + 2) % 3

                def wait_buf(s2=s2):
                    out_wait(s2)

                if par == 0:
                    pl.when(k > 0)(wait_buf)
                else:
                    wait_buf()
                iwait(s2)
                g1_start(s2)
                lwait(s1)
                build_il(k + 1, s1)
                g1_wait(s1)
                g2_start(s1)
                g2_wait(s0)
                compute(s0)
                out_start(k, s0)
                ifetch_start(k + 3, s0)
            return 0

        lax.fori_loop(0, niter, body, 0)

        out_wait(klast % 3)
        g2_wait((klast + 1) % 3)
        g1_wait((klast + 2) % 3)
        iwait(klast % 3)
        lwait((klast + 2) % 3)
        lwait(klast % 3)

    return sc_kernel


def kernel(sequence, segment_label, token_table, seg_table, gamma, beta, pe):
    B, S = sequence.shape
    V = token_table.shape[0]
    seq = sequence.reshape(-1).astype(jnp.int32)
    lab = segment_label.reshape(-1).astype(jnp.int32)
    comb = (seg_table[:, None, :] + pe[None, :S, :]).reshape(NSEG * S, DIM)
    out = _build(B, S, V)(seq, lab, token_table, comb, gamma, beta)
    return out.reshape(B, S, DIM)


# speedup vs baseline: 2.1332x; 1.0336x over previous
import functools

import jax
import jax.numpy as jnp
from jax import lax
from jax.experimental import pallas as pl
from jax.experimental.pallas import tpu as pltpu
from jax.experimental.pallas import tpu_sc as plsc

DIM = 128
NSEG = 3
EPS = 1e-5
LANES = 16
NJ = DIM // LANES
NC = 2
NS = 16
NW = NC * NS
C = 128


def _build(B, S, V):
    rows_total = B * S
    rows_per_tile = rows_total // NW
    nchunk = rows_per_tile // C
    chunks_per_seq = S // C
    inv_d = 1.0 / DIM
    niter = (nchunk + 1 + 2) // 3
    klast = niter * 3 - 1

    mesh = plsc.VectorSubcoreMesh(core_axis_name="c", subcore_axis_name="s")

    @functools.partial(
        pl.kernel,
        mesh=mesh,
        out_type=jax.ShapeDtypeStruct((rows_total, DIM), jnp.float32),
        compiler_params=pltpu.CompilerParams(needs_layout_passes=False),
        scratch_types=(
            [pltpu.VMEM((DIM,), jnp.float32)] * 2
            + [pltpu.VMEM((C,), jnp.int32)] * 3
            + [pltpu.VMEM((C,), jnp.int32)] * 3
            + [pltpu.VMEM((C,), jnp.int32)] * 3
            + [pltpu.VMEM((C, DIM), jnp.float32)] * 3
            + [pltpu.VMEM_SHARED((NSEG * S, DIM), jnp.float32)]
            + [pltpu.SemaphoreType.DMA] * 15
        ),
    )
    def sc_kernel(seq_hbm, lab_hbm, tok_hbm, comb_hbm, gamma_hbm, beta_hbm,
                  out_hbm, gamma_v, beta_v, idx0, idx1, idx2, lab0, lab1,
                  lab2, il0, il1, il2, buf0, buf1, buf2, comb_sp, isem0,
                  isem1, isem2,
                  lsem0, lsem1, lsem2, g1s0, g1s1, g1s2, g2s0, g2s1, g2s2,
                  osem0, osem1, osem2):
        wid = lax.axis_index("s") * NC + lax.axis_index("c")
        row_base = wid * rows_per_tile
        lane = lax.iota(jnp.int32, LANES)
        idx = [idx0, idx1, idx2]
        labs = [lab0, lab1, lab2]
        ils = [il0, il1, il2]
        buf = [buf0, buf1, buf2]
        isem = [isem0, isem1, isem2]
        lsem = [lsem0, lsem1, lsem2]
        g1sem = [g1s0, g1s1, g1s2]
        g2sem = [g2s0, g2s1, g2s2]
        osem = [osem0, osem1, osem2]

        @pl.when(lax.axis_index("s") == 0)
        def _stage_comb():
            pltpu.sync_copy(comb_hbm, comb_sp)

        plsc.subcore_barrier()
        pltpu.sync_copy(gamma_hbm, gamma_v)
        pltpu.sync_copy(beta_hbm, beta_v)
        gammas = [gamma_v[pl.ds(j * LANES, LANES)] for j in range(NJ)]
        betas = [beta_v[pl.ds(j * LANES, LANES)] for j in range(NJ)]

        def chunk_base(k):
            return row_base + jnp.minimum(k, nchunk - 1) * C

        def chunk_p0(k):
            return lax.rem(jnp.minimum(k, nchunk - 1), chunks_per_seq) * C

        def ifetch_start(k, slot):
            base = chunk_base(k)
            pltpu.make_async_copy(seq_hbm.at[pl.ds(base, C)], idx[slot],
                                  isem[slot]).start()
            pltpu.make_async_copy(lab_hbm.at[pl.ds(base, C)], labs[slot],
                                  lsem[slot]).start()

        def iwait(slot):
            pltpu.make_async_copy(seq_hbm.at[pl.ds(row_base, C)], idx[slot],
                                  isem[slot]).wait()

        def lwait(slot):
            pltpu.make_async_copy(lab_hbm.at[pl.ds(row_base, C)], labs[slot],
                                  lsem[slot]).wait()

        def build_il(k, slot):
            p0 = chunk_p0(k)
            lab_s = labs[slot]
            il_s = ils[slot]
            for g in range(C // LANES):
                lv = lab_s[pl.ds(g * LANES, LANES)]
                il_s[pl.ds(g * LANES, LANES)] = lv * S + (p0 + g * LANES + lane)

        def g1_start(slot):
            pltpu.make_async_copy(tok_hbm.at[idx[slot]], buf[slot],
                                  g1sem[slot]).start()

        def g1_wait(slot):
            pltpu.make_async_copy(tok_hbm.at[idx[slot]], buf[slot],
                                  g1sem[slot]).wait()

        def g2_start(slot):
            pltpu.make_async_copy(comb_sp.at[ils[slot]], buf[slot],
                                  g2sem[slot]).start(add=True)

        def g2_wait(slot):
            pltpu.make_async_copy(comb_sp.at[ils[slot]], buf[slot],
                                  g2sem[slot]).wait()

        def out_start(k, slot):
            pltpu.make_async_copy(buf[slot], out_hbm.at[pl.ds(chunk_base(k), C)],
                                  osem[slot]).start()

        def out_wait(slot):
            pltpu.make_async_copy(buf[slot], out_hbm.at[pl.ds(row_base, C)],
                                  osem[slot]).wait()

        def compute(slot):
            buf_s = buf[slot]

            @plsc.parallel_loop(0, C, unroll=1)
            def row_body(r):
                s = None
                q = None
                xs = []
                for j in range(NJ):
                    x = buf_s[r, pl.ds(j * LANES, LANES)]
                    xs.append(x)
                    s = x if s is None else s + x
                    q = x * x if q is None else q + x * x
                mean = jnp.full((LANES,), jnp.sum(s), jnp.float32) * inv_d
                var = (jnp.full((LANES,), jnp.sum(q), jnp.float32) * inv_d
                       - mean * mean)
                ve = var + EPS
                seed = jnp.int32(0x5F3759DF) - (plsc.bitcast(ve, jnp.int32) >> 1)
                y = plsc.bitcast(seed, jnp.float32)
                for _ in range(2):
                    y = y * (1.5 - 0.5 * ve * y * y)
                for j in range(NJ):
                    out = (xs[j] - mean) * y * gammas[j] + betas[j]
                    buf_s[r, pl.ds(j * LANES, LANES)] = out

        ifetch_start(0, 0)
        ifetch_start(1, 1)
        ifetch_start(2, 2)
        iwait(0)
        g1_start(0)
        lwait(0)
        build_il(0, 0)
        iwait(1)
        g1_start(1)
        g1_wait(0)
        g2_start(0)

        def body(k3, _):
            for par in range(3):
                k = k3 * 3 + par
                s0 = par
                s1 = (par + 1) % 3
                s2 = (par + 2) % 3

                def wait_buf(s2=s2):
                    out_wait(s2)

                if par == 0:
                    pl.when(k > 0)(wait_buf)
                else:
                    wait_buf()
                iwait(s2)
                g1_start(s2)
                lwait(s1)
                build_il(k + 1, s1)
                g1_wait(s1)
                g2_start(s1)
                g2_wait(s0)
                compute(s0)
                out_start(k, s0)
                ifetch_start(k + 3, s0)
            return 0

        lax.fori_loop(0, niter, body, 0)

        out_wait(klast % 3)
        g2_wait((klast + 1) % 3)
        g1_wait((klast + 2) % 3)
        iwait(klast % 3)
        lwait((klast + 2) % 3)
        lwait(klast % 3)

    return sc_kernel


def kernel(sequence, segment_label, token_table, seg_table, gamma, beta, pe):
    B, S = sequence.shape
    V = token_table.shape[0]
    seq = sequence.reshape(-1).astype(jnp.int32)
    lab = segment_label.reshape(-1).astype(jnp.int32)
    comb = (seg_table[:, None, :] + pe[None, :S, :]).reshape(NSEG * S, DIM)
    out = _build(B, S, V)(seq, lab, token_table, comb, gamma, beta)
    return out.reshape(B, S, DIM)
